# baseline (speedup 1.0000x reference)
import dataclasses
import functools

import jax
import jax.numpy as jnp
from jax import lax
from jax.experimental import pallas as pl
from jax.experimental.pallas import tpu as pltpu
from jax.experimental.pallas import tpu_sc as plsc

N_NODES = 10000
N_EDGES = 160000
D = 256
DH = D // 2

NC = 2
NS = 16
LANES = 16
N_PAD = 10240
E_PAD = 163840
E_PER_SUB = E_PAD // NS
CHUNK = 128
NCHUNK = E_PER_SUB // CHUNK
SUPER = 8
NSUPER = NCHUNK // SUPER
ROWS_PER_SUB = N_PAD // NS
WB_ROWS = 624
HROWS = 80
NZ = ROWS_PER_SUB // CHUNK
ZTAIL = ROWS_PER_SUB - NZ * CHUNK


def _sc_aggregate(h_stack, srcs, dsts):
    mesh = plsc.VectorSubcoreMesh(core_axis_name="c", subcore_axis_name="s")

    cp = pltpu.CompilerParams()
    if "needs_layout_passes" in pltpu.CompilerParams.__dataclass_fields__:
        cp = dataclasses.replace(cp, needs_layout_passes=False)

    @functools.partial(
        pl.kernel,
        compiler_params=cp,
        out_type=(
            jax.ShapeDtypeStruct((NC, N_NODES, DH), jnp.float32),
            jax.ShapeDtypeStruct((NC, HROWS, 128), jnp.float32),
        ),
        mesh=mesh,
        scratch_types=[
            pltpu.VMEM((SUPER, CHUNK), jnp.int32),
            pltpu.VMEM((SUPER, CHUNK), jnp.int32),
            pltpu.VMEM((SUPER, CHUNK), jnp.int32),
            pltpu.VMEM((SUPER, CHUNK), jnp.int32),
            pltpu.VMEM((CHUNK, DH), jnp.float32),
            pltpu.VMEM((CHUNK, DH), jnp.float32),
            pltpu.VMEM((HROWS, 128), jnp.float32),
            pltpu.VMEM((HROWS,), jnp.int32),
            pltpu.VMEM_SHARED((N_PAD, DH), jnp.float32),
            pltpu.VMEM_SHARED((HROWS, 128), jnp.float32),
            pltpu.SemaphoreType.DMA,
            pltpu.SemaphoreType.DMA,
            pltpu.SemaphoreType.DMA,
            pltpu.SemaphoreType.DMA,
            pltpu.SemaphoreType.DMA,
        ],
    )
    def agg(hs_hbm, srcs_hbm, dsts_hbm, sums_hbm, cnts_hbm,
            src0_v, src1_v, dst0_v, dst1_v, rows_a, rows_b, hist_v, lin_v,
            acc_s, deg_s, sem_ga, sem_gb, sem_sa, sem_sb, sem_idx):
        src_slots = (src0_v, src1_v)
        dst_slots = (dst0_v, dst1_v)
        c = lax.axis_index("c")
        s = lax.axis_index("s")

        zvec = jnp.zeros((LANES,), jnp.float32)
        hvec = jnp.full((LANES,), 0.5, jnp.float32)

        @pl.loop(0, CHUNK)
        def _(r):
            for cc in range(DH // LANES):
                rows_a[r, pl.ds(cc * LANES, LANES)] = zvec

        @pl.loop(0, HROWS)
        def _(r):
            for cc in range(128 // LANES):
                hist_v[r, pl.ds(cc * LANES, LANES)] = zvec
        for k in range(HROWS // LANES):
            lin_v[pl.ds(k * LANES, LANES)] = lax.iota(jnp.int32, 16) + k * LANES

        @pl.loop(0, NZ)
        def _(k):
            pltpu.sync_copy(rows_a,
                            acc_s.at[pl.ds(s * ROWS_PER_SUB + k * CHUNK, CHUNK)])
        if ZTAIL:
            pltpu.sync_copy(rows_a.at[pl.ds(0, ZTAIL)],
                            acc_s.at[pl.ds(s * ROWS_PER_SUB + NZ * CHUNK,
                                           ZTAIL)])
        pltpu.sync_copy(rows_a.at[pl.ds(0, HROWS // NS)],
                        deg_s.at[pl.ds(s * (HROWS // NS), HROWS // NS)])

        pltpu.sync_copy(srcs_hbm.at[c, s, 0], src0_v)
        pltpu.sync_copy(dsts_hbm.at[s, 0], dst0_v)
        pltpu.async_copy(hs_hbm.at[src0_v.at[0]], rows_a, sem_ga)
        pltpu.async_copy(hs_hbm.at[src0_v.at[1]], rows_b, sem_gb)

        plsc.subcore_barrier()

        def hist_update(slot, i):
            for k in range(CHUNK // LANES):
                d16 = dst_slots[slot][i, pl.ds(k * LANES, LANES)]
                plsc.addupdate_scatter(
                    hist_v,
                    [lax.shift_right_logical(d16, 7),
                     lax.bitwise_and(d16, 127)],
                    hvec)

        BUFS = ((rows_a, sem_ga, sem_sa), (rows_b, sem_gb, sem_sb))

        def drain_scatter(buf_ix):
            rbuf, _, ssem = BUFS[buf_ix]
            pltpu.make_async_copy(rbuf, acc_s.at[dst_slots[0].at[0]],
                                  ssem).wait()

        def step(sw, jl, buf_ix, issue_ahead):
            rbuf, gsem, ssem = BUFS[buf_ix]
            pltpu.make_async_copy(
                hs_hbm.at[src_slots[sw].at[jl]], rbuf, gsem).wait()
            pltpu.async_copy(rbuf, acc_s.at[dst_slots[sw].at[jl]], ssem,
                             add=True)
            hist_update(sw, jl)
            if issue_ahead:
                @pl.when(jl + 2 <= SUPER - 1)
                def _():
                    drain_scatter(buf_ix)
                    pltpu.async_copy(hs_hbm.at[src_slots[sw].at[jl + 2]],
                                     rbuf, gsem)

        for w in range(NSUPER):
            sw = w % 2
            b0 = w % 2
            if w + 1 < NSUPER:
                pltpu.async_copy(srcs_hbm.at[c, s, w + 1], src_slots[1 - sw],
                                 sem_idx)
                pltpu.async_copy(dsts_hbm.at[s, w + 1], dst_slots[1 - sw],
                                 sem_idx)

            @pl.loop(0, SUPER // 2)
            def _(p, sw=sw, b0=b0):
                step(sw, 2 * p, b0, True)
                step(sw, 2 * p + 1, 1 - b0, True)

            if w + 1 < NSUPER:
                pltpu.make_async_copy(srcs_hbm.at[c, s, w + 1],
                                      src_slots[1 - sw], sem_idx).wait()
                pltpu.make_async_copy(dsts_hbm.at[s, w + 1],
                                      dst_slots[1 - sw], sem_idx).wait()
                drain_scatter(1 - b0)
                pltpu.async_copy(hs_hbm.at[src_slots[1 - sw].at[0]],
                                 BUFS[1 - b0][0], BUFS[1 - b0][1])
                drain_scatter(b0)
                pltpu.async_copy(hs_hbm.at[src_slots[1 - sw].at[1]],
                                 BUFS[b0][0], BUFS[b0][1])

        drain_scatter(0)
        drain_scatter(1)
        pltpu.sync_copy(hist_v, deg_s.at[lin_v], add=True)

        plsc.subcore_barrier()

        off = s * WB_ROWS
        pltpu.sync_copy(acc_s.at[pl.ds(off, WB_ROWS)],
                        sums_hbm.at[c, pl.ds(off, WB_ROWS)])

        @pl.when(s == NS - 1)
        def _():
            tail = NS * WB_ROWS
            pltpu.sync_copy(acc_s.at[pl.ds(tail, N_NODES - tail)],
                            sums_hbm.at[c, pl.ds(tail, N_NODES - tail)])

        @pl.when(s < HROWS // 8)
        def _():
            pltpu.sync_copy(deg_s.at[pl.ds(s * 8, 8)],
                            cnts_hbm.at[c, pl.ds(s * 8, 8)])

    return agg(h_stack, srcs, dsts)


BLK = 1000


def _finish(sums, cnts):
    deg_halves = jnp.reshape(cnts, (NC, HROWS * 128))[:, :N_NODES, None]

    def body(s_ref, c_ref, o_ref):
        deg = c_ref[0] + c_ref[1]
        norm = 1.0 / jnp.maximum(deg, 1.0)
        o_ref[:, :DH] = jnp.maximum(s_ref[0] * norm, 0.0)
        o_ref[:, DH:] = jnp.maximum(s_ref[1] * norm, 0.0)

    return pl.pallas_call(
        body,
        grid=(N_NODES // BLK,),
        in_specs=[
            pl.BlockSpec((NC, BLK, DH), lambda i: (0, i, 0)),
            pl.BlockSpec((NC, BLK, 1), lambda i: (0, i, 0)),
        ],
        out_specs=pl.BlockSpec((BLK, D), lambda i: (i, 0)),
        out_shape=jax.ShapeDtypeStruct((N_NODES, D), jnp.float32),
    )(sums, deg_halves)


def kernel(h, edge_index):
    src = edge_index[0]
    dst = edge_index[1]
    pad_e = E_PAD - N_EDGES
    src = jnp.concatenate([src, jnp.zeros((pad_e,), src.dtype)])
    dst = jnp.concatenate(
        [dst,
         N_NODES + (jnp.arange(pad_e, dtype=dst.dtype) % (N_PAD - N_NODES))])
    h_stack = jnp.reshape(h, (2 * N_NODES, DH))
    src2 = src * 2
    srcs = jnp.stack([src2, src2 + 1]).reshape(NC, NS, NSUPER, SUPER, CHUNK)
    dsts = dst.reshape(NS, NSUPER, SUPER, CHUNK)
    sums, cnts = _sc_aggregate(h_stack, srcs, dsts)
    return _finish(sums, cnts)


# speedup vs baseline: 2.0910x; 2.0910x over previous
import dataclasses
import functools

import jax
import jax.numpy as jnp
from jax import lax
from jax.experimental import pallas as pl
from jax.experimental.pallas import tpu as pltpu
from jax.experimental.pallas import tpu_sc as plsc

N_NODES = 10000
N_EDGES = 160000
D = 256
DH = D // 2

NC = 2
NS = 16
LANES = 16
E_PER_SUB = N_EDGES // NS
CHUNK = 80
NCHUNK = E_PER_SUB // CHUNK
SUPER = 25
NSUPER = NCHUNK // SUPER
NPAIR = (NCHUNK - 1) // 2
ROWS_PER_SUB = N_NODES // NS
WB_ROWS = 624
HROWS = 80


def _sc_aggregate(h_stack, srcs, dsts):
    mesh = plsc.VectorSubcoreMesh(core_axis_name="c", subcore_axis_name="s")

    cp = pltpu.CompilerParams()
    if "needs_layout_passes" in pltpu.CompilerParams.__dataclass_fields__:
        cp = dataclasses.replace(cp, needs_layout_passes=False)

    @functools.partial(
        pl.kernel,
        compiler_params=cp,
        out_type=(
            jax.ShapeDtypeStruct((NC, N_NODES, DH), jnp.float32),
            jax.ShapeDtypeStruct((NC, HROWS, 128), jnp.float32),
        ),
        mesh=mesh,
        scratch_types=[
            pltpu.VMEM((SUPER, CHUNK), jnp.int32),
            pltpu.VMEM((SUPER, CHUNK), jnp.int32),
            pltpu.VMEM((SUPER, CHUNK), jnp.int32),
            pltpu.VMEM((SUPER, CHUNK), jnp.int32),
            pltpu.VMEM((CHUNK, DH), jnp.float32),
            pltpu.VMEM((CHUNK, DH), jnp.float32),
            pltpu.VMEM((HROWS, 128), jnp.float32),
            pltpu.VMEM((HROWS,), jnp.int32),
            pltpu.VMEM_SHARED((N_NODES, DH), jnp.float32),
            pltpu.VMEM_SHARED((HROWS, 128), jnp.float32),
            pltpu.SemaphoreType.DMA,
            pltpu.SemaphoreType.DMA,
            pltpu.SemaphoreType.DMA,
            pltpu.SemaphoreType.DMA,
            pltpu.SemaphoreType.DMA,
        ],
    )
    def agg(hs_hbm, srcs_hbm, dsts_hbm, sums_hbm, cnts_hbm,
            src0_v, src1_v, dst0_v, dst1_v, rows_a, rows_b, hist_v, lin_v,
            acc_s, deg_s, sem_ga, sem_gb, sem_sa, sem_sb, sem_idx):
        src_slots = (src0_v, src1_v)
        dst_slots = (dst0_v, dst1_v)
        c = lax.axis_index("c")
        s = lax.axis_index("s")

        zvec = jnp.zeros((LANES,), jnp.float32)
        hvec = jnp.full((LANES,), 0.5, jnp.float32)

        @pl.loop(0, CHUNK)
        def _(r):
            for cc in range(DH // LANES):
                rows_a[r, pl.ds(cc * LANES, LANES)] = zvec
            for cc in range(128 // LANES):
                hist_v[r, pl.ds(cc * LANES, LANES)] = zvec
        for k in range(HROWS // LANES):
            lin_v[pl.ds(k * LANES, LANES)] = lax.iota(jnp.int32, 16) + k * LANES

        @pl.loop(0, 7)
        def _(k):
            pltpu.sync_copy(rows_a,
                            acc_s.at[pl.ds(s * ROWS_PER_SUB + k * CHUNK, CHUNK)])
        pltpu.sync_copy(rows_a.at[pl.ds(0, ROWS_PER_SUB - 7 * CHUNK)],
                        acc_s.at[pl.ds(s * ROWS_PER_SUB + 7 * CHUNK,
                                       ROWS_PER_SUB - 7 * CHUNK)])
        pltpu.sync_copy(rows_a.at[pl.ds(0, HROWS // NS)],
                        deg_s.at[pl.ds(s * (HROWS // NS), HROWS // NS)])

        pltpu.sync_copy(srcs_hbm.at[c, s, 0], src0_v)
        pltpu.sync_copy(dsts_hbm.at[s, 0], dst0_v)
        pltpu.async_copy(hs_hbm.at[src0_v.at[0]], rows_a, sem_ga)
        pltpu.async_copy(hs_hbm.at[src0_v.at[1]], rows_b, sem_gb)

        plsc.subcore_barrier()

        def hist_update(slot, i):
            for k in range(CHUNK // LANES):
                d16 = dst_slots[slot][i, pl.ds(k * LANES, LANES)]
                plsc.addupdate_scatter(
                    hist_v,
                    [lax.shift_right_logical(d16, 7),
                     lax.bitwise_and(d16, 127)],
                    hvec)

        BUFS = ((rows_a, sem_ga, sem_sa), (rows_b, sem_gb, sem_sb))

        def drain_scatter(buf_ix):
            rbuf, _, ssem = BUFS[buf_ix]
            pltpu.make_async_copy(rbuf, acc_s.at[dst_slots[0].at[0]],
                                  ssem).wait()

        def step(sw, jl, buf_ix, issue_ahead):
            rbuf, gsem, ssem = BUFS[buf_ix]
            pltpu.make_async_copy(
                hs_hbm.at[src_slots[sw].at[jl]], rbuf, gsem).wait()
            pltpu.async_copy(rbuf, acc_s.at[dst_slots[sw].at[jl]], ssem,
                             add=True)
            hist_update(sw, jl)
            if issue_ahead:
                @pl.when(jl + 2 <= SUPER - 1)
                def _():
                    drain_scatter(buf_ix)
                    pltpu.async_copy(hs_hbm.at[src_slots[sw].at[jl + 2]],
                                     rbuf, gsem)

        for w in range(NSUPER):
            sw = w % 2
            b0 = w % 2
            if w + 1 < NSUPER:
                pltpu.async_copy(srcs_hbm.at[c, s, w + 1], src_slots[1 - sw],
                                 sem_idx)
                pltpu.async_copy(dsts_hbm.at[s, w + 1], dst_slots[1 - sw],
                                 sem_idx)

            @pl.loop(0, (SUPER - 1) // 2)
            def _(p, sw=sw, b0=b0):
                step(sw, 2 * p, b0, True)
                step(sw, 2 * p + 1, 1 - b0, True)

            step(sw, SUPER - 1, b0, False)

            if w + 1 < NSUPER:
                pltpu.make_async_copy(srcs_hbm.at[c, s, w + 1],
                                      src_slots[1 - sw], sem_idx).wait()
                pltpu.make_async_copy(dsts_hbm.at[s, w + 1],
                                      dst_slots[1 - sw], sem_idx).wait()
                drain_scatter(1 - b0)
                pltpu.async_copy(hs_hbm.at[src_slots[1 - sw].at[0]],
                                 BUFS[1 - b0][0], BUFS[1 - b0][1])
                drain_scatter(b0)
                pltpu.async_copy(hs_hbm.at[src_slots[1 - sw].at[1]],
                                 BUFS[b0][0], BUFS[b0][1])

        drain_scatter(0)
        drain_scatter(1)
        pltpu.sync_copy(hist_v, deg_s.at[lin_v], add=True)

        plsc.subcore_barrier()

        off = s * WB_ROWS
        pltpu.sync_copy(acc_s.at[pl.ds(off, WB_ROWS)],
                        sums_hbm.at[c, pl.ds(off, WB_ROWS)])

        @pl.when(s == NS - 1)
        def _():
            tail = NS * WB_ROWS
            pltpu.sync_copy(acc_s.at[pl.ds(tail, N_NODES - tail)],
                            sums_hbm.at[c, pl.ds(tail, N_NODES - tail)])

        @pl.when(s < HROWS // 8)
        def _():
            pltpu.sync_copy(deg_s.at[pl.ds(s * 8, 8)],
                            cnts_hbm.at[c, pl.ds(s * 8, 8)])

    return agg(h_stack, srcs, dsts)


BLK = 1000


def _finish(sums, cnts):
    deg_halves = jnp.reshape(cnts, (NC, HROWS * 128))[:, :N_NODES, None]

    def body(s_ref, c_ref, o_ref):
        deg = c_ref[0] + c_ref[1]
        norm = 1.0 / jnp.maximum(deg, 1.0)
        o_ref[:, :DH] = jnp.maximum(s_ref[0] * norm, 0.0)
        o_ref[:, DH:] = jnp.maximum(s_ref[1] * norm, 0.0)

    return pl.pallas_call(
        body,
        grid=(N_NODES // BLK,),
        in_specs=[
            pl.BlockSpec((NC, BLK, DH), lambda i: (0, i, 0)),
            pl.BlockSpec((NC, BLK, 1), lambda i: (0, i, 0)),
        ],
        out_specs=pl.BlockSpec((BLK, D), lambda i: (i, 0)),
        out_shape=jax.ShapeDtypeStruct((N_NODES, D), jnp.float32),
    )(sums, deg_halves)


def kernel(h, edge_index):
    src = edge_index[0]
    dst = edge_index[1]
    h_stack = jnp.reshape(h, (2 * N_NODES, DH))
    src2 = src * 2
    srcs = jnp.stack([src2, src2 + 1]).reshape(NC, NS, NSUPER, SUPER, CHUNK)
    dsts = dst.reshape(NS, NSUPER, SUPER, CHUNK)
    sums, cnts = _sc_aggregate(h_stack, srcs, dsts)
    return _finish(sums, cnts)
